# Initial kernel scaffold; baseline (speedup 1.0000x reference)
import functools

import jax
import jax.numpy as jnp
import numpy as np
from jax import lax
from jax.experimental import pallas as pl
from jax.experimental.pallas import tpu as pltpu
from jax.experimental.pallas import tpu_sc as plsc

_B, _C, _H, _W = 16, 3, 512, 512
_N_OUT = _B * _C * _H * _W
_N_LAT = 16 * 192 * 32 * 32

_NC, _NS, _L = 2, 16, 16
_NW = _NC * _NS
_CHUNK = 8192
_PW_O = _N_OUT // _NW
_PW_L = _N_LAT // _NW
_NCH_O = _PW_O // _CHUNK
_NCH_L = _PW_L // _CHUNK
_HBINS = 256
_HSIZE = _HBINS * _L


def _minmax_body(lat_ref, out_ref, mn_ref, mx_ref):
    i = pl.program_id(0)
    x = lat_ref[...]
    bmn = jnp.min(x)
    bmx = jnp.max(x)

    @pl.when(i == 0)
    def _():
        mn_ref[0] = bmn
        mx_ref[0] = bmx

    @pl.when(i > 0)
    def _():
        mn_ref[0] = jnp.minimum(mn_ref[0], bmn)
        mx_ref[0] = jnp.maximum(mx_ref[0], bmx)

    @pl.when(i == pl.num_programs(0) - 1)
    def _():
        vmin = mn_ref[0]
        inv = 256.0 / (mx_ref[0] - vmin)
        out_ref[...] = jnp.concatenate(
            [jnp.full((1, _L), vmin), jnp.full((1, _L), inv)], axis=0
        )


_minmax = pl.pallas_call(
    _minmax_body,
    grid=(6,),
    in_specs=[pl.BlockSpec((512, 1024), lambda i: (i, 0))],
    out_specs=pl.BlockSpec((2, _L), lambda i: (0, 0)),
    out_shape=jax.ShapeDtypeStruct((2, _L), jnp.float32),
    scratch_shapes=[
        pltpu.SMEM((1,), jnp.float32),
        pltpu.SMEM((1,), jnp.float32),
    ],
)


def _mse_body(o_ref, i_ref, out_ref, acc_ref):
    i = pl.program_id(0)
    d = o_ref[...] - i_ref[...]
    s = jnp.sum(d * d)

    @pl.when(i == 0)
    def _():
        acc_ref[0] = s

    @pl.when(i > 0)
    def _():
        acc_ref[0] = acc_ref[0] + s

    @pl.when(i == pl.num_programs(0) - 1)
    def _():
        out_ref[0, 0] = acc_ref[0] / float(_N_OUT)


_mse = pl.pallas_call(
    _mse_body,
    grid=(6,),
    in_specs=[
        pl.BlockSpec((2048, 1024), lambda i: (i, 0)),
        pl.BlockSpec((2048, 1024), lambda i: (i, 0)),
    ],
    out_specs=pl.BlockSpec(memory_space=pltpu.SMEM),
    out_shape=jax.ShapeDtypeStruct((1, 1), jnp.float32),
    scratch_shapes=[pltpu.SMEM((1,), jnp.float32)],
)


def _sc_hist_body(data_hbm, lat_hbm, mm_hbm, out_hbm,
                  buf0, buf1, hist_o, hist_l, vmin_r, vinv_r, sem0, sem1):
    wid = lax.axis_index("s") * _NC + lax.axis_index("c")
    zero16 = jnp.zeros((_L,), jnp.float32)

    def zbody(i, c):
        hist_o[pl.ds(i * _L, _L)] = zero16
        hist_l[pl.ds(i * _L, _L)] = zero16
        return c

    lax.fori_loop(0, _HSIZE // _L, zbody, 0, unroll=4)

    pltpu.sync_copy(mm_hbm.at[0], vmin_r)
    pltpu.sync_copy(mm_hbm.at[1], vinv_r)
    vminv = vmin_r[...]
    vinvv = vinv_r[...]

    lane_base = lax.iota(jnp.int32, _L) * _HBINS
    ones = jnp.ones((_L,), jnp.float32)
    c255 = jnp.full((_L,), 255.0, jnp.float32)
    chalf = jnp.full((_L,), 0.5, jnp.float32)
    c255i = jnp.full((_L,), _HBINS - 1, jnp.int32)

    bufs = (buf0, buf1)
    sems = (sem0, sem1)

    def stream(src_hbm, base, nch, process_group):
        copies = {}

        def start(g):
            b = g & 1
            copies[g] = pltpu.async_copy(
                src_hbm.at[pl.ds(base + g * _CHUNK, _CHUNK)], bufs[b], sems[b]
            )

        start(0)
        for g in range(nch):
            if g + 1 < nch:
                start(g + 1)
            copies[g].wait()
            buf = bufs[g & 1]

            def gbody(i, c):
                v = buf[pl.ds(i * _L, _L)]
                process_group(v)
                return c

            lax.fori_loop(0, _CHUNK // _L, gbody, 0, unroll=8)

    def pg_out(v):
        q = (v * c255 + chalf).astype(jnp.int32)
        plsc.addupdate_scatter(hist_o, [q + lane_base], ones)

    def pg_lat(v):
        q = ((v - vminv) * vinvv).astype(jnp.int32)
        q = jnp.minimum(q, c255i)
        plsc.addupdate_scatter(hist_l, [q + lane_base], ones)

    stream(data_hbm, wid * _PW_O, _NCH_O, pg_out)
    stream(lat_hbm, wid * _PW_L, _NCH_L, pg_lat)

    pltpu.sync_copy(hist_o, out_hbm.at[wid, 0])
    pltpu.sync_copy(hist_l, out_hbm.at[wid, 1])


_sc_hist = pl.kernel(
    _sc_hist_body,
    out_type=jax.ShapeDtypeStruct((_NW, 2, _HSIZE), jnp.float32),
    mesh=plsc.VectorSubcoreMesh(
        core_axis_name="c", subcore_axis_name="s",
        num_cores=_NC, num_subcores=_NS,
    ),
    scratch_types=[
        pltpu.VMEM((_CHUNK,), jnp.float32),
        pltpu.VMEM((_CHUNK,), jnp.float32),
        pltpu.VMEM((_HSIZE,), jnp.float32),
        pltpu.VMEM((_HSIZE,), jnp.float32),
        pltpu.VMEM((_L,), jnp.float32),
        pltpu.VMEM((_L,), jnp.float32),
        pltpu.SemaphoreType.DMA,
        pltpu.SemaphoreType.DMA,
    ],
)


def _finalize_body(ho_ref, hl_ref, dist_ref, loss_ref, bpp_ref, dout_ref, ent_ref):
    inv_ln2 = 1.0 / float(np.log(2.0))

    def entropy(h2):
        h = jnp.sum(h2, axis=0, keepdims=True)
        tot = jnp.sum(h)
        p = jnp.clip(h / tot, 1e-12, 1.0)
        return -jnp.sum(p * (jnp.log(p) * inv_ln2))

    ent_o = entropy(ho_ref[...])
    ent_l = entropy(hl_ref[...]) / float(_B)
    dist = dist_ref[0, 0]
    loss_ref[0, 0] = dist + ent_l
    bpp_ref[0, 0] = ent_o * float(_C) / float(_H * _W)
    dout_ref[0, 0] = dist
    ent_ref[0, 0] = ent_l


_finalize = pl.pallas_call(
    _finalize_body,
    in_specs=[
        pl.BlockSpec(memory_space=pltpu.ANY),
        pl.BlockSpec(memory_space=pltpu.ANY),
        pl.BlockSpec(memory_space=pltpu.SMEM),
    ],
    out_specs=[pl.BlockSpec(memory_space=pltpu.SMEM)] * 4,
    out_shape=[jax.ShapeDtypeStruct((1, 1), jnp.float32)] * 4,
)


def kernel(outputs, inputs, latent):
    out2d = outputs.reshape(_N_OUT // 1024, 1024)
    in2d = inputs.reshape(_N_OUT // 1024, 1024)
    lat2d = latent.reshape(_N_LAT // 1024, 1024)

    mm = _minmax(lat2d)
    hists = _sc_hist(outputs.reshape(-1), latent.reshape(-1), mm)
    dist = _mse(out2d, in2d)
    ho2 = hists[:, 0, :].reshape(_NW * _L, _HBINS)
    hl2 = hists[:, 1, :].reshape(_NW * _L, _HBINS)
    loss, bpp, dout, ent = _finalize(ho2, hl2, dist)
    return (loss[0, 0], bpp[0, 0], dout[0, 0], ent[0, 0])


# speedup vs baseline: 1.2875x; 1.2875x over previous
import functools

import jax
import jax.numpy as jnp
import numpy as np
from jax import lax
from jax.experimental import pallas as pl
from jax.experimental.pallas import tpu as pltpu
from jax.experimental.pallas import tpu_sc as plsc

_B, _C, _H, _W = 16, 3, 512, 512
_N_OUT = _B * _C * _H * _W
_N_LAT = 16 * 192 * 32 * 32

_NC, _NS, _L = 2, 16, 16
_NW = _NC * _NS
_CHUNK = 8192
_PW_O = _N_OUT // _NW
_PW_L = _N_LAT // _NW
_NCH_O = _PW_O // _CHUNK
_NCH_L = _PW_L // _CHUNK
_HBINS = 256
_HSIZE = _HBINS * _L


def _minmax_body(lat_ref, out_ref, mn_ref, mx_ref):
    i = pl.program_id(0)
    x = lat_ref[...]
    bmn = jnp.min(x)
    bmx = jnp.max(x)

    @pl.when(i == 0)
    def _():
        mn_ref[0] = bmn
        mx_ref[0] = bmx

    @pl.when(i > 0)
    def _():
        mn_ref[0] = jnp.minimum(mn_ref[0], bmn)
        mx_ref[0] = jnp.maximum(mx_ref[0], bmx)

    @pl.when(i == pl.num_programs(0) - 1)
    def _():
        vmin = mn_ref[0]
        inv = 256.0 / (mx_ref[0] - vmin)
        out_ref[...] = jnp.concatenate(
            [jnp.full((1, _L), vmin), jnp.full((1, _L), inv)], axis=0
        )


_minmax = pl.pallas_call(
    _minmax_body,
    grid=(6,),
    in_specs=[pl.BlockSpec((512, 1024), lambda i: (i, 0))],
    out_specs=pl.BlockSpec((2, _L), lambda i: (0, 0)),
    out_shape=jax.ShapeDtypeStruct((2, _L), jnp.float32),
    scratch_shapes=[
        pltpu.SMEM((1,), jnp.float32),
        pltpu.SMEM((1,), jnp.float32),
    ],
)


def _mse_body(o_ref, i_ref, out_ref, acc_ref):
    i = pl.program_id(0)
    d = o_ref[...] - i_ref[...]
    s = jnp.sum(d * d)

    @pl.when(i == 0)
    def _():
        acc_ref[0] = s

    @pl.when(i > 0)
    def _():
        acc_ref[0] = acc_ref[0] + s

    @pl.when(i == pl.num_programs(0) - 1)
    def _():
        out_ref[0, 0] = acc_ref[0] / float(_N_OUT)


_mse = pl.pallas_call(
    _mse_body,
    grid=(6,),
    in_specs=[
        pl.BlockSpec((2048, 1024), lambda i: (i, 0)),
        pl.BlockSpec((2048, 1024), lambda i: (i, 0)),
    ],
    out_specs=pl.BlockSpec(memory_space=pltpu.SMEM),
    out_shape=jax.ShapeDtypeStruct((1, 1), jnp.float32),
    scratch_shapes=[pltpu.SMEM((1,), jnp.float32)],
)


def _sc_hist_body(data_hbm, lat_hbm, mm_hbm, out_hbm,
                  buf0, buf1, hist_o, hist_l, vmin_r, vinv_r, sem0, sem1):
    wid = lax.axis_index("s") * _NC + lax.axis_index("c")
    zero16 = jnp.zeros((_L,), jnp.float32)

    def zbody(i, c):
        hist_o[pl.ds(i * _L, _L)] = zero16
        hist_l[pl.ds(i * _L, _L)] = zero16
        return c

    lax.fori_loop(0, _HSIZE // _L, zbody, 0, unroll=4)

    pltpu.sync_copy(mm_hbm.at[0], vmin_r)
    pltpu.sync_copy(mm_hbm.at[1], vinv_r)
    vminv = vmin_r[...]
    vinvv = vinv_r[...]

    lane_base = lax.iota(jnp.int32, _L) * _HBINS
    ones = jnp.ones((_L,), jnp.float32)
    c255 = jnp.full((_L,), 255.0, jnp.float32)
    chalf = jnp.full((_L,), 0.5, jnp.float32)
    c255i = jnp.full((_L,), _HBINS - 1, jnp.int32)

    bufs = (buf0, buf1)
    sems = (sem0, sem1)

    def stream(src_hbm, base, nch, process_group):
        copies = {}

        def start(g):
            b = g & 1
            copies[g] = pltpu.async_copy(
                src_hbm.at[pl.ds(base + g * _CHUNK, _CHUNK)], bufs[b], sems[b]
            )

        start(0)
        for g in range(nch):
            if g + 1 < nch:
                start(g + 1)
            copies[g].wait()
            buf = bufs[g & 1]

            def gbody(i, c):
                v = buf[pl.ds(i * _L, _L)]
                process_group(v)
                return c

            lax.fori_loop(0, _CHUNK // _L, gbody, 0, unroll=8)

    def pg_out(v):
        q = (v * c255 + chalf).astype(jnp.int32)
        plsc.addupdate_scatter(hist_o, [q + lane_base], ones)

    def pg_lat(v):
        q = ((v - vminv) * vinvv).astype(jnp.int32)
        q = jnp.minimum(q, c255i)
        plsc.addupdate_scatter(hist_l, [q + lane_base], ones)

    stream(data_hbm, wid * _PW_O, _NCH_O, pg_out)
    stream(lat_hbm, wid * _PW_L, _NCH_L, pg_lat)

    pltpu.sync_copy(hist_o, out_hbm.at[wid, 0])
    pltpu.sync_copy(hist_l, out_hbm.at[wid, 1])


@functools.cache
def _get_sc_hist():
    return pl.kernel(
        _sc_hist_body,
        out_type=jax.ShapeDtypeStruct((_NW, 2, _HSIZE), jnp.float32),
        mesh=plsc.VectorSubcoreMesh(
            core_axis_name="c", subcore_axis_name="s",
            num_cores=_NC, num_subcores=_NS,
        ),
        scratch_types=[
        pltpu.VMEM((_CHUNK,), jnp.float32),
        pltpu.VMEM((_CHUNK,), jnp.float32),
        pltpu.VMEM((_HSIZE,), jnp.float32),
        pltpu.VMEM((_HSIZE,), jnp.float32),
            pltpu.VMEM((_L,), jnp.float32),
            pltpu.VMEM((_L,), jnp.float32),
            pltpu.SemaphoreType.DMA,
            pltpu.SemaphoreType.DMA,
        ],
        compiler_params=pltpu.CompilerParams(needs_layout_passes=False),
    )


def _finalize_body(ho_ref, hl_ref, dist_ref, loss_ref, bpp_ref, dout_ref, ent_ref):
    inv_ln2 = 1.0 / float(np.log(2.0))

    def entropy(h2):
        h = jnp.sum(h2, axis=0, keepdims=True)
        tot = jnp.sum(h)
        p = jnp.clip(h / tot, 1e-12, 1.0)
        return -jnp.sum(p * (jnp.log(p) * inv_ln2))

    ent_o = entropy(ho_ref[...])
    ent_l = entropy(hl_ref[...]) / float(_B)
    dist = dist_ref[0, 0]
    loss_ref[0, 0] = dist + ent_l
    bpp_ref[0, 0] = ent_o * float(_C) / float(_H * _W)
    dout_ref[0, 0] = dist
    ent_ref[0, 0] = ent_l


_finalize = pl.pallas_call(
    _finalize_body,
    in_specs=[
        pl.BlockSpec(),
        pl.BlockSpec(),
        pl.BlockSpec(memory_space=pltpu.SMEM),
    ],
    out_specs=[pl.BlockSpec(memory_space=pltpu.SMEM)] * 4,
    out_shape=[jax.ShapeDtypeStruct((1, 1), jnp.float32)] * 4,
)


def kernel(outputs, inputs, latent):
    out2d = outputs.reshape(_N_OUT // 1024, 1024)
    in2d = inputs.reshape(_N_OUT // 1024, 1024)
    lat2d = latent.reshape(_N_LAT // 1024, 1024)

    mm = _minmax(lat2d)
    hists = _get_sc_hist()(outputs.reshape(-1), latent.reshape(-1), mm)
    dist = _mse(out2d, in2d)
    ho2 = hists[:, 0, :].reshape(_NW * _L, _HBINS)
    hl2 = hists[:, 1, :].reshape(_NW * _L, _HBINS)
    loss, bpp, dout, ent = _finalize(ho2, hl2, dist)
    return (loss[0, 0], bpp[0, 0], dout[0, 0], ent[0, 0])


# speedup vs baseline: 2.3776x; 1.8467x over previous
import functools

import jax
import jax.numpy as jnp
import numpy as np
from jax import lax
from jax.experimental import pallas as pl
from jax.experimental.pallas import tpu as pltpu
from jax.experimental.pallas import tpu_sc as plsc

_B, _C, _H, _W = 16, 3, 512, 512
_N_OUT = _B * _C * _H * _W
_N_LAT = 16 * 192 * 32 * 32

_NC, _NS, _L = 2, 16, 16
_NW = _NC * _NS
_CHUNK = 8192
_PW_O = _N_OUT // _NW
_PW_L = _N_LAT // _NW
_NCH_O = _PW_O // _CHUNK
_NCH_L = _PW_L // _CHUNK
_HBINS = 256
_HSIZE = _HBINS * _L


def _minmax_body(lat_ref, out_ref, mn_ref, mx_ref):
    i = pl.program_id(0)
    x = lat_ref[...]
    bmn = jnp.min(x)
    bmx = jnp.max(x)

    @pl.when(i == 0)
    def _():
        mn_ref[0] = bmn
        mx_ref[0] = bmx

    @pl.when(i > 0)
    def _():
        mn_ref[0] = jnp.minimum(mn_ref[0], bmn)
        mx_ref[0] = jnp.maximum(mx_ref[0], bmx)

    @pl.when(i == pl.num_programs(0) - 1)
    def _():
        vmin = mn_ref[0]
        inv = 256.0 / (mx_ref[0] - vmin)
        out_ref[...] = jnp.concatenate(
            [jnp.full((1, _L), vmin), jnp.full((1, _L), inv)], axis=0
        )


_minmax = pl.pallas_call(
    _minmax_body,
    grid=(6,),
    in_specs=[pl.BlockSpec((512, 1024), lambda i: (i, 0))],
    out_specs=pl.BlockSpec((2, _L), lambda i: (0, 0)),
    out_shape=jax.ShapeDtypeStruct((2, _L), jnp.float32),
    scratch_shapes=[
        pltpu.SMEM((1,), jnp.float32),
        pltpu.SMEM((1,), jnp.float32),
    ],
)


def _mse_body(o_ref, i_ref, out_ref, acc_ref):
    i = pl.program_id(0)
    d = o_ref[...] - i_ref[...]
    s = jnp.sum(d * d)

    @pl.when(i == 0)
    def _():
        acc_ref[0] = s

    @pl.when(i > 0)
    def _():
        acc_ref[0] = acc_ref[0] + s

    @pl.when(i == pl.num_programs(0) - 1)
    def _():
        out_ref[0, 0] = acc_ref[0] / float(_N_OUT)


_mse = pl.pallas_call(
    _mse_body,
    grid=(6,),
    in_specs=[
        pl.BlockSpec((2048, 1024), lambda i: (i, 0)),
        pl.BlockSpec((2048, 1024), lambda i: (i, 0)),
    ],
    out_specs=pl.BlockSpec(memory_space=pltpu.SMEM),
    out_shape=jax.ShapeDtypeStruct((1, 1), jnp.float32),
    scratch_shapes=[pltpu.SMEM((1,), jnp.float32)],
)


def _sc_hist_body(data_hbm, lat_hbm, mm_hbm, out_hbm,
                  buf0, buf1, hist_o, hist_l, vmin_r, vinv_r, sem0, sem1):
    wid = lax.axis_index("s") * _NC + lax.axis_index("c")
    zero16 = jnp.zeros((_L,), jnp.float32)

    def zbody(i, c):
        hist_o[pl.ds(i * _L, _L)] = zero16
        hist_l[pl.ds(i * _L, _L)] = zero16
        return c

    lax.fori_loop(0, _HSIZE // _L, zbody, 0, unroll=4)

    pltpu.sync_copy(mm_hbm.at[0], vmin_r)
    pltpu.sync_copy(mm_hbm.at[1], vinv_r)
    vminv = vmin_r[...]
    vinvv = vinv_r[...]

    lane_base = lax.iota(jnp.int32, _L) * _HBINS
    ones = jnp.ones((_L,), jnp.float32)
    c255 = jnp.full((_L,), 255.0, jnp.float32)
    chalf = jnp.full((_L,), 0.5, jnp.float32)
    c255i = jnp.full((_L,), _HBINS - 1, jnp.int32)

    bufs = (buf0, buf1)
    sems = (sem0, sem1)

    def stream(src_hbm, base, nch, process_group):
        copies = {}

        def start(g):
            b = g & 1
            copies[g] = pltpu.async_copy(
                src_hbm.at[pl.ds(base + g * _CHUNK, _CHUNK)], bufs[b], sems[b]
            )

        start(0)
        for g in range(nch):
            if g + 1 < nch:
                start(g + 1)
            copies[g].wait()
            buf = bufs[g & 1]

            @plsc.parallel_loop(0, _CHUNK // _L, unroll=8)
            def _(i):
                v = buf[pl.ds(i * _L, _L)]
                process_group(v)

    def pg_out(v):
        q = (v * c255 + chalf).astype(jnp.int32)
        plsc.addupdate_scatter(hist_o, [q + lane_base], ones)

    def pg_lat(v):
        q = ((v - vminv) * vinvv).astype(jnp.int32)
        q = jnp.minimum(q, c255i)
        plsc.addupdate_scatter(hist_l, [q + lane_base], ones)

    stream(data_hbm, wid * _PW_O, _NCH_O, pg_out)
    stream(lat_hbm, wid * _PW_L, _NCH_L, pg_lat)

    pltpu.sync_copy(hist_o, out_hbm.at[wid, 0])
    pltpu.sync_copy(hist_l, out_hbm.at[wid, 1])


@functools.cache
def _get_sc_hist():
    return pl.kernel(
        _sc_hist_body,
        out_type=jax.ShapeDtypeStruct((_NW, 2, _HSIZE), jnp.float32),
        mesh=plsc.VectorSubcoreMesh(
            core_axis_name="c", subcore_axis_name="s",
            num_cores=_NC, num_subcores=_NS,
        ),
        scratch_types=[
        pltpu.VMEM((_CHUNK,), jnp.float32),
        pltpu.VMEM((_CHUNK,), jnp.float32),
        pltpu.VMEM((_HSIZE,), jnp.float32),
        pltpu.VMEM((_HSIZE,), jnp.float32),
            pltpu.VMEM((_L,), jnp.float32),
            pltpu.VMEM((_L,), jnp.float32),
            pltpu.SemaphoreType.DMA,
            pltpu.SemaphoreType.DMA,
        ],
        compiler_params=pltpu.CompilerParams(needs_layout_passes=False),
    )


def _finalize_body(ho_ref, hl_ref, dist_ref, loss_ref, bpp_ref, dout_ref, ent_ref):
    inv_ln2 = 1.0 / float(np.log(2.0))

    def entropy(h2):
        h = jnp.sum(h2, axis=0, keepdims=True)
        tot = jnp.sum(h)
        p = jnp.clip(h / tot, 1e-12, 1.0)
        return -jnp.sum(p * (jnp.log(p) * inv_ln2))

    ent_o = entropy(ho_ref[...])
    ent_l = entropy(hl_ref[...]) / float(_B)
    dist = dist_ref[0, 0]
    loss_ref[0, 0] = dist + ent_l
    bpp_ref[0, 0] = ent_o * float(_C) / float(_H * _W)
    dout_ref[0, 0] = dist
    ent_ref[0, 0] = ent_l


_finalize = pl.pallas_call(
    _finalize_body,
    in_specs=[
        pl.BlockSpec(),
        pl.BlockSpec(),
        pl.BlockSpec(memory_space=pltpu.SMEM),
    ],
    out_specs=[pl.BlockSpec(memory_space=pltpu.SMEM)] * 4,
    out_shape=[jax.ShapeDtypeStruct((1, 1), jnp.float32)] * 4,
)


def kernel(outputs, inputs, latent):
    out2d = outputs.reshape(_N_OUT // 1024, 1024)
    in2d = inputs.reshape(_N_OUT // 1024, 1024)
    lat2d = latent.reshape(_N_LAT // 1024, 1024)

    mm = _minmax(lat2d)
    hists = _get_sc_hist()(outputs.reshape(-1), latent.reshape(-1), mm)
    dist = _mse(out2d, in2d)
    ho2 = hists[:, 0, :].reshape(_NW * _L, _HBINS)
    hl2 = hists[:, 1, :].reshape(_NW * _L, _HBINS)
    loss, bpp, dout, ent = _finalize(ho2, hl2, dist)
    return (loss[0, 0], bpp[0, 0], dout[0, 0], ent[0, 0])


# speedup vs baseline: 3.1860x; 1.3400x over previous
import functools

import jax
import jax.numpy as jnp
import numpy as np
from jax import lax
from jax.experimental import pallas as pl
from jax.experimental.pallas import tpu as pltpu
from jax.experimental.pallas import tpu_sc as plsc

_B, _C, _H, _W = 16, 3, 512, 512
_N_OUT = _B * _C * _H * _W
_N_LAT = 16 * 192 * 32 * 32

_NC, _NS, _L = 2, 16, 16
_NW = _NC * _NS
_CHUNK = 16384
_PW_O = _N_OUT // _NW
_PW_L = _N_LAT // _NW
_NCH_O = _PW_O // _CHUNK
_NCH_L = _PW_L // _CHUNK
_HBINS = 256
_HSIZE = _HBINS * _L


def _minmax_body(lat_ref, out_ref, mn_ref, mx_ref):
    i = pl.program_id(0)
    x = lat_ref[...]
    bmn = jnp.min(x)
    bmx = jnp.max(x)

    @pl.when(i == 0)
    def _():
        mn_ref[0] = bmn
        mx_ref[0] = bmx

    @pl.when(i > 0)
    def _():
        mn_ref[0] = jnp.minimum(mn_ref[0], bmn)
        mx_ref[0] = jnp.maximum(mx_ref[0], bmx)

    @pl.when(i == pl.num_programs(0) - 1)
    def _():
        vmin = mn_ref[0]
        inv = 256.0 / (mx_ref[0] - vmin)
        out_ref[...] = jnp.concatenate(
            [jnp.full((1, _L), vmin), jnp.full((1, _L), inv)], axis=0
        )


_minmax = pl.pallas_call(
    _minmax_body,
    grid=(4,),
    in_specs=[pl.BlockSpec((4, 192, 32, 32), lambda i: (i, 0, 0, 0))],
    out_specs=pl.BlockSpec((2, _L), lambda i: (0, 0)),
    out_shape=jax.ShapeDtypeStruct((2, _L), jnp.float32),
    scratch_shapes=[
        pltpu.SMEM((1,), jnp.float32),
        pltpu.SMEM((1,), jnp.float32),
    ],
)


def _mse_body(o_ref, i_ref, out_ref, acc_ref):
    i = pl.program_id(0)
    d = o_ref[...] - i_ref[...]
    s = jnp.sum(d * d)

    @pl.when(i == 0)
    def _():
        acc_ref[0] = s

    @pl.when(i > 0)
    def _():
        acc_ref[0] = acc_ref[0] + s

    @pl.when(i == pl.num_programs(0) - 1)
    def _():
        out_ref[0, 0] = acc_ref[0] / float(_N_OUT)


_mse = pl.pallas_call(
    _mse_body,
    grid=(8,),
    in_specs=[
        pl.BlockSpec((2, 3, 512, 512), lambda i: (i, 0, 0, 0)),
        pl.BlockSpec((2, 3, 512, 512), lambda i: (i, 0, 0, 0)),
    ],
    out_specs=pl.BlockSpec(memory_space=pltpu.SMEM),
    out_shape=jax.ShapeDtypeStruct((1, 1), jnp.float32),
    scratch_shapes=[pltpu.SMEM((1,), jnp.float32)],
)


def _sc_hist_body(data_hbm, lat_hbm, mm_hbm, out_hbm,
                  buf0, buf1, hist_o, hist_l, vmin_r, vinv_r, sem0, sem1):
    wid = lax.axis_index("s") * _NC + lax.axis_index("c")
    zero16 = jnp.zeros((_L,), jnp.float32)

    def zbody(i, c):
        hist_o[pl.ds(i * _L, _L)] = zero16
        hist_l[pl.ds(i * _L, _L)] = zero16
        return c

    lax.fori_loop(0, _HSIZE // _L, zbody, 0, unroll=4)

    pltpu.sync_copy(mm_hbm.at[0], vmin_r)
    pltpu.sync_copy(mm_hbm.at[1], vinv_r)
    vminv = vmin_r[...]
    vinvv = vinv_r[...]

    lane_base = lax.iota(jnp.int32, _L) * _HBINS
    ones = jnp.ones((_L,), jnp.float32)
    c255 = jnp.full((_L,), 255.0, jnp.float32)
    chalf = jnp.full((_L,), 0.5, jnp.float32)
    c255i = jnp.full((_L,), _HBINS - 1, jnp.int32)

    bufs = (buf0, buf1)
    sems = (sem0, sem1)

    def stream(src_hbm, base, nch, process_group):
        copies = {}

        def start(g):
            b = g & 1
            copies[g] = pltpu.async_copy(
                src_hbm.at[pl.ds(base + g * _CHUNK, _CHUNK)], bufs[b], sems[b]
            )

        start(0)
        for g in range(nch):
            if g + 1 < nch:
                start(g + 1)
            copies[g].wait()
            buf = bufs[g & 1]

            @plsc.parallel_loop(0, _CHUNK // _L, unroll=8)
            def _(i):
                v = buf[pl.ds(i * _L, _L)]
                process_group(v)

    def pg_out(v):
        q = (v * c255 + chalf).astype(jnp.int32)
        plsc.addupdate_scatter(hist_o, [q + lane_base], ones)

    def pg_lat(v):
        q = ((v - vminv) * vinvv).astype(jnp.int32)
        q = jnp.minimum(q, c255i)
        plsc.addupdate_scatter(hist_l, [q + lane_base], ones)

    stream(data_hbm, wid * _PW_O, _NCH_O, pg_out)
    stream(lat_hbm, wid * _PW_L, _NCH_L, pg_lat)

    pltpu.sync_copy(hist_o, out_hbm.at[wid, 0])
    pltpu.sync_copy(hist_l, out_hbm.at[wid, 1])


@functools.cache
def _get_sc_hist():
    return pl.kernel(
        _sc_hist_body,
        out_type=jax.ShapeDtypeStruct((_NW, 2, _HSIZE), jnp.float32),
        mesh=plsc.VectorSubcoreMesh(
            core_axis_name="c", subcore_axis_name="s",
            num_cores=_NC, num_subcores=_NS,
        ),
        scratch_types=[
        pltpu.VMEM((_CHUNK,), jnp.float32),
        pltpu.VMEM((_CHUNK,), jnp.float32),
        pltpu.VMEM((_HSIZE,), jnp.float32),
        pltpu.VMEM((_HSIZE,), jnp.float32),
            pltpu.VMEM((_L,), jnp.float32),
            pltpu.VMEM((_L,), jnp.float32),
            pltpu.SemaphoreType.DMA,
            pltpu.SemaphoreType.DMA,
        ],
        compiler_params=pltpu.CompilerParams(needs_layout_passes=False),
    )


def _finalize_body(ho_ref, hl_ref, dist_ref, loss_ref, bpp_ref, dout_ref, ent_ref):
    inv_ln2 = 1.0 / float(np.log(2.0))

    def entropy(h2):
        h = jnp.sum(h2, axis=0, keepdims=True)
        tot = jnp.sum(h)
        p = jnp.clip(h / tot, 1e-12, 1.0)
        return -jnp.sum(p * (jnp.log(p) * inv_ln2))

    ent_o = entropy(ho_ref[...])
    ent_l = entropy(hl_ref[...]) / float(_B)
    dist = dist_ref[0, 0]
    loss_ref[0, 0] = dist + ent_l
    bpp_ref[0, 0] = ent_o * float(_C) / float(_H * _W)
    dout_ref[0, 0] = dist
    ent_ref[0, 0] = ent_l


_finalize = pl.pallas_call(
    _finalize_body,
    in_specs=[
        pl.BlockSpec(),
        pl.BlockSpec(),
        pl.BlockSpec(memory_space=pltpu.SMEM),
    ],
    out_specs=[pl.BlockSpec(memory_space=pltpu.SMEM)] * 4,
    out_shape=[jax.ShapeDtypeStruct((1, 1), jnp.float32)] * 4,
)


def kernel(outputs, inputs, latent):
    mm = _minmax(latent)
    hists = _get_sc_hist()(outputs.reshape(-1), latent.reshape(-1), mm)
    dist = _mse(outputs, inputs)
    ho2 = hists[:, 0, :].reshape(_NW * _L, _HBINS)
    hl2 = hists[:, 1, :].reshape(_NW * _L, _HBINS)
    loss, bpp, dout, ent = _finalize(ho2, hl2, dist)
    return (loss[0, 0], bpp[0, 0], dout[0, 0], ent[0, 0])


# speedup vs baseline: 4.1435x; 1.3005x over previous
import functools

import jax
import jax.numpy as jnp
import numpy as np
from jax import lax
from jax.experimental import pallas as pl
from jax.experimental.pallas import tpu as pltpu
from jax.experimental.pallas import tpu_sc as plsc

_B, _C, _H, _W = 16, 3, 512, 512
_N_OUT = _B * _C * _H * _W
_N_LAT = 16 * 192 * 32 * 32

_NC, _NS, _L = 2, 16, 16
_NW = _NC * _NS
_CHUNK = 16384
_PW_O = _N_OUT // _NW
_PW_L = _N_LAT // _NW
_NCH_O = _PW_O // _CHUNK
_NCH_L = _PW_L // _CHUNK
_HBINS = 256
_HSIZE = _HBINS * _L


def _minmax_body(lat_ref, out_ref, mn_ref, mx_ref):
    i = pl.program_id(0)
    x = lat_ref[...]
    bmn = jnp.min(x)
    bmx = jnp.max(x)

    @pl.when(i == 0)
    def _():
        mn_ref[0] = bmn
        mx_ref[0] = bmx

    @pl.when(i > 0)
    def _():
        mn_ref[0] = jnp.minimum(mn_ref[0], bmn)
        mx_ref[0] = jnp.maximum(mx_ref[0], bmx)

    @pl.when(i == pl.num_programs(0) - 1)
    def _():
        vmin = mn_ref[0]
        inv = 256.0 / (mx_ref[0] - vmin)
        out_ref[...] = jnp.concatenate(
            [jnp.full((1, _L), vmin), jnp.full((1, _L), inv)], axis=0
        )


_minmax = pl.pallas_call(
    _minmax_body,
    grid=(4,),
    in_specs=[pl.BlockSpec((4, 32, 32, 192), lambda i: (i, 0, 0, 0))],
    out_specs=pl.BlockSpec((2, _L), lambda i: (0, 0)),
    out_shape=jax.ShapeDtypeStruct((2, _L), jnp.float32),
    scratch_shapes=[
        pltpu.SMEM((1,), jnp.float32),
        pltpu.SMEM((1,), jnp.float32),
    ],
)


def _mse_body(o_ref, i_ref, out_ref, acc_ref):
    i = pl.program_id(0)
    d = o_ref[...] - i_ref[...]
    s = jnp.sum(d * d)

    @pl.when(i == 0)
    def _():
        acc_ref[0] = s

    @pl.when(i > 0)
    def _():
        acc_ref[0] = acc_ref[0] + s

    @pl.when(i == pl.num_programs(0) - 1)
    def _():
        out_ref[0, 0] = acc_ref[0] / float(_N_OUT)


_mse = pl.pallas_call(
    _mse_body,
    grid=(8,),
    in_specs=[
        pl.BlockSpec((2, 3, 512, 512), lambda i: (i, 0, 0, 0)),
        pl.BlockSpec((2, 3, 512, 512), lambda i: (i, 0, 0, 0)),
    ],
    out_specs=pl.BlockSpec(memory_space=pltpu.SMEM),
    out_shape=jax.ShapeDtypeStruct((1, 1), jnp.float32),
    scratch_shapes=[pltpu.SMEM((1,), jnp.float32)],
)


def _sc_hist_body(data_hbm, lat_hbm, mm_hbm, out_hbm,
                  buf0, buf1, hist_o, hist_l, vmin_r, vinv_r, sem0, sem1):
    wid = lax.axis_index("s") * _NC + lax.axis_index("c")
    zero16 = jnp.zeros((_L,), jnp.float32)

    def zbody(i, c):
        hist_o[pl.ds(i * _L, _L)] = zero16
        hist_l[pl.ds(i * _L, _L)] = zero16
        return c

    lax.fori_loop(0, _HSIZE // _L, zbody, 0, unroll=4)

    pltpu.sync_copy(mm_hbm.at[0], vmin_r)
    pltpu.sync_copy(mm_hbm.at[1], vinv_r)
    vminv = vmin_r[...]
    vinvv = vinv_r[...]

    lane_base = lax.iota(jnp.int32, _L) * _HBINS
    ones = jnp.ones((_L,), jnp.float32)
    c255 = jnp.full((_L,), 255.0, jnp.float32)
    chalf = jnp.full((_L,), 0.5, jnp.float32)
    c255i = jnp.full((_L,), _HBINS - 1, jnp.int32)

    bufs = (buf0, buf1)
    sems = (sem0, sem1)

    def stream(src_hbm, base, nch, process_group):
        copies = {}

        def start(g):
            b = g & 1
            copies[g] = pltpu.async_copy(
                src_hbm.at[pl.ds(base + g * _CHUNK, _CHUNK)], bufs[b], sems[b]
            )

        start(0)
        for g in range(nch):
            if g + 1 < nch:
                start(g + 1)
            copies[g].wait()
            buf = bufs[g & 1]

            @plsc.parallel_loop(0, _CHUNK // _L, unroll=16)
            def _(i):
                v = buf[pl.ds(i * _L, _L)]
                process_group(v)

    def pg_out(v):
        q = (v * c255 + chalf).astype(jnp.int32)
        plsc.addupdate_scatter(hist_o, [q + lane_base], ones)

    def pg_lat(v):
        q = ((v - vminv) * vinvv).astype(jnp.int32)
        q = jnp.minimum(q, c255i)
        plsc.addupdate_scatter(hist_l, [q + lane_base], ones)

    stream(data_hbm, wid * _PW_O, _NCH_O, pg_out)
    stream(lat_hbm, wid * _PW_L, _NCH_L, pg_lat)

    pltpu.sync_copy(hist_o, out_hbm.at[wid, 0])
    pltpu.sync_copy(hist_l, out_hbm.at[wid, 1])


@functools.cache
def _get_sc_hist():
    return pl.kernel(
        _sc_hist_body,
        out_type=jax.ShapeDtypeStruct((_NW, 2, _HSIZE), jnp.float32),
        mesh=plsc.VectorSubcoreMesh(
            core_axis_name="c", subcore_axis_name="s",
            num_cores=_NC, num_subcores=_NS,
        ),
        scratch_types=[
        pltpu.VMEM((_CHUNK,), jnp.float32),
        pltpu.VMEM((_CHUNK,), jnp.float32),
        pltpu.VMEM((_HSIZE,), jnp.float32),
        pltpu.VMEM((_HSIZE,), jnp.float32),
            pltpu.VMEM((_L,), jnp.float32),
            pltpu.VMEM((_L,), jnp.float32),
            pltpu.SemaphoreType.DMA,
            pltpu.SemaphoreType.DMA,
        ],
        compiler_params=pltpu.CompilerParams(needs_layout_passes=False),
    )


def _finalize_body(ho_ref, hl_ref, dist_ref, loss_ref, bpp_ref, dout_ref, ent_ref):
    inv_ln2 = 1.0 / float(np.log(2.0))

    def entropy(h2):
        h = jnp.sum(h2, axis=0, keepdims=True)
        tot = jnp.sum(h)
        p = jnp.clip(h / tot, 1e-12, 1.0)
        return -jnp.sum(p * (jnp.log(p) * inv_ln2))

    ent_o = entropy(ho_ref[...])
    ent_l = entropy(hl_ref[...]) / float(_B)
    dist = dist_ref[0, 0]
    loss_ref[0, 0] = dist + ent_l
    bpp_ref[0, 0] = ent_o * float(_C) / float(_H * _W)
    dout_ref[0, 0] = dist
    ent_ref[0, 0] = ent_l


_finalize = pl.pallas_call(
    _finalize_body,
    in_specs=[
        pl.BlockSpec(),
        pl.BlockSpec(),
        pl.BlockSpec(memory_space=pltpu.SMEM),
    ],
    out_specs=[pl.BlockSpec(memory_space=pltpu.SMEM)] * 4,
    out_shape=[jax.ShapeDtypeStruct((1, 1), jnp.float32)] * 4,
)


def kernel(outputs, inputs, latent):
    lat = jnp.transpose(latent, (0, 2, 3, 1))
    mm = _minmax(lat)
    hists = _get_sc_hist()(outputs.reshape(-1), lat.reshape(-1), mm)
    dist = _mse(outputs, inputs)
    ho2 = hists[:, 0, :].reshape(_NW * _L, _HBINS)
    hl2 = hists[:, 1, :].reshape(_NW * _L, _HBINS)
    loss, bpp, dout, ent = _finalize(ho2, hl2, dist)
    return (loss[0, 0], bpp[0, 0], dout[0, 0], ent[0, 0])
